# Initial kernel scaffold; baseline (speedup 1.0000x reference)
import jax
import jax.numpy as jnp
from jax.experimental import pallas as pl


def kernel(x_nchw, w2d, b2d):
    raise NotImplementedError("write your pallas kernel here")


# speedup vs baseline: 14.9580x; 14.9580x over previous
import functools

import numpy as np
import jax
import jax.numpy as jnp
from jax import lax
from jax.experimental import pallas as pl
from jax.experimental.pallas import tpu as pltpu


def _pool_matrices(th, wo):
    w2 = 2 * wo
    pint = np.zeros((2 * th * w2, th * wo), np.float32)
    for y in range(th):
        for xo in range(wo):
            base = 2 * y * w2 + 2 * xo
            for l in (base, base + 1, base + w2, base + w2 + 1):
                pint[l, y * wo + xo] = 0.25
    phalo = np.zeros((2 * w2, wo), np.float32)
    for xo in range(wo):
        base = 2 * xo
        for l in (base, base + 1, base + w2, base + w2 + 1):
            phalo[l, xo] = 0.25
    return jnp.asarray(pint), jnp.asarray(phalo)


def _pool_conv_kernel(x_ref, w_ref, b_ref, pint_ref, phalo_ref, o_ref,
                      flat_ref, *, cin, th, wo):
    hw = x_ref.shape[2]
    w2 = 2 * wo
    nt = th * wo
    lanes = 2 * th * w2
    r = pl.program_id(1)
    nr = pl.num_programs(1)
    cdt = flat_ref.dtype

    s0 = pl.multiple_of(r * lanes, lanes)
    t0 = pl.multiple_of(jnp.maximum(s0 - 2 * w2, 0), 2 * w2)
    b0 = pl.multiple_of(jnp.minimum(s0 + lanes, hw - 2 * w2), 2 * w2)

    slab = x_ref[0, :, pl.ds(s0, lanes)].astype(cdt)
    interior = jnp.dot(slab, pint_ref[...], preferred_element_type=jnp.float32)
    top = jnp.dot(x_ref[0, :, pl.ds(t0, 2 * w2)].astype(cdt), phalo_ref[...],
                  preferred_element_type=jnp.float32)
    bot = jnp.dot(x_ref[0, :, pl.ds(b0, 2 * w2)].astype(cdt), phalo_ref[...],
                  preferred_element_type=jnp.float32)
    top = top * (r > 0).astype(top.dtype)
    bot = bot * (r < nr - 1).astype(bot.dtype)

    zc = jnp.zeros((cin, 1), cdt)
    flat_ref[:, 0:1] = zc
    flat_ref[:, 1:1 + wo] = top.astype(cdt)
    flat_ref[:, 1 + wo:1 + wo + nt] = interior.astype(cdt)
    flat_ref[:, 1 + wo + nt:1 + 2 * wo + nt] = bot.astype(cdt)
    flat_ref[:, 1 + 2 * wo + nt:2 + 2 * wo + nt] = zc

    col = lax.broadcasted_iota(jnp.int32, (1, th, wo), 2).reshape(1, nt)

    acc = b_ref[...] * jnp.ones((1, nt), jnp.float32)
    for kh in range(3):
        for kw in range(3):
            tap = kh * 3 + kw
            win = flat_ref[:, kh * wo + kw:kh * wo + kw + nt]
            if kw == 0:
                win = jnp.where(col != 0, win, jnp.zeros_like(win))
            elif kw == 2:
                win = jnp.where(col != wo - 1, win, jnp.zeros_like(win))
            acc = acc + jnp.dot(w_ref[:, tap * cin:(tap + 1) * cin], win,
                                preferred_element_type=jnp.float32)
    o_ref[0] = acc.astype(o_ref.dtype)


def kernel(x_nchw, w2d, b2d):
    n, cin, h, w = x_nchw.shape
    cout = w2d.shape[0]
    ho, wo = h // 2, w // 2
    th = 8 if ho % 8 == 0 else ho
    nt = th * wo
    rt = ho // th

    pint, phalo = _pool_matrices(th, wo)
    pint = pint.astype(jnp.bfloat16)
    phalo = phalo.astype(jnp.bfloat16)
    body = functools.partial(_pool_conv_kernel, cin=cin, th=th, wo=wo)
    out = pl.pallas_call(
        body,
        out_shape=jax.ShapeDtypeStruct((n, cout, ho * wo), x_nchw.dtype),
        grid=(n, rt),
        in_specs=[
            pl.BlockSpec((1, cin, h * w), lambda i, k: (i, 0, 0)),
            pl.BlockSpec((cout, 9 * cin), lambda i, k: (0, 0)),
            pl.BlockSpec((cout, 1), lambda i, k: (0, 0)),
            pl.BlockSpec(pint.shape, lambda i, k: (0, 0)),
            pl.BlockSpec(phalo.shape, lambda i, k: (0, 0)),
        ],
        out_specs=pl.BlockSpec((1, cout, nt), lambda i, k: (i, 0, k)),
        scratch_shapes=[
            pltpu.VMEM((cin, (th + 2) * wo + 2), jnp.bfloat16),
        ],
        compiler_params=pltpu.CompilerParams(
            dimension_semantics=("parallel", "parallel"),
            vmem_limit_bytes=48 * 1024 * 1024,
        ),
    )(x_nchw.reshape(n, cin, h * w), w2d.astype(jnp.bfloat16),
      b2d.astype(jnp.float32), pint, phalo)
    return out.reshape(n, cout, ho, wo)


# speedup vs baseline: 18.6381x; 1.2460x over previous
import functools

import numpy as np
import jax
import jax.numpy as jnp
from jax import lax
from jax.experimental import pallas as pl
from jax.experimental.pallas import tpu as pltpu


def _pool_matrices(tq, wo):
    w2 = 2 * wo
    pint = np.zeros((2 * tq * w2, tq * wo), np.float32)
    for y in range(tq):
        for xo in range(wo):
            base = 2 * y * w2 + 2 * xo
            for l in (base, base + 1, base + w2, base + w2 + 1):
                pint[l, y * wo + xo] = 0.25
    phalo = np.zeros((2 * w2, wo), np.float32)
    for xo in range(wo):
        base = 2 * xo
        for l in (base, base + 1, base + w2, base + w2 + 1):
            phalo[l, xo] = 0.25
    return jnp.asarray(pint), jnp.asarray(phalo)


def _pool_conv_kernel(x_ref, w_ref, b_ref, pint_ref, phalo_ref, o_ref,
                      flat_ref, *, cin, th, tq, wo):
    hw = x_ref.shape[2]
    w2 = 2 * wo
    nt = th * wo
    nq = tq * wo
    lanes_t = 2 * th * w2
    lanes_q = 2 * tq * w2
    r = pl.program_id(1)
    nr = pl.num_programs(1)
    cdt = flat_ref.dtype

    s0 = pl.multiple_of(r * lanes_t, lanes_t)
    t0 = pl.multiple_of(jnp.maximum(s0 - 2 * w2, 0), 2 * w2)
    b0 = pl.multiple_of(jnp.minimum(s0 + lanes_t, hw - 2 * w2), 2 * w2)

    for q in range(th // tq):
        slab = x_ref[0, :, pl.ds(s0 + q * lanes_q, lanes_q)]
        part = jnp.dot(slab, pint_ref[...], preferred_element_type=jnp.float32)
        flat_ref[:, 1 + wo + q * nq:1 + wo + (q + 1) * nq] = part.astype(cdt)

    top = jnp.dot(x_ref[0, :, pl.ds(t0, 2 * w2)], phalo_ref[...],
                  preferred_element_type=jnp.float32)
    bot = jnp.dot(x_ref[0, :, pl.ds(b0, 2 * w2)], phalo_ref[...],
                  preferred_element_type=jnp.float32)
    top = top * (r > 0).astype(top.dtype)
    bot = bot * (r < nr - 1).astype(bot.dtype)

    zc = jnp.zeros((cin, 1), cdt)
    flat_ref[:, 0:1] = zc
    flat_ref[:, 1:1 + wo] = top.astype(cdt)
    flat_ref[:, 1 + wo + nt:1 + 2 * wo + nt] = bot.astype(cdt)
    flat_ref[:, 1 + 2 * wo + nt:2 + 2 * wo + nt] = zc

    col = lax.broadcasted_iota(jnp.int32, (1, th, wo), 2).reshape(1, nt)
    acc = b_ref[...] * jnp.ones((1, nt), jnp.float32)
    for kh in range(3):
        for kw in range(3):
            tap = kh * 3 + kw
            win = flat_ref[:, kh * wo + kw:kh * wo + kw + nt]
            if kw == 0:
                win = jnp.where(col != 0, win, jnp.zeros_like(win))
            elif kw == 2:
                win = jnp.where(col != wo - 1, win, jnp.zeros_like(win))
            acc = acc + jnp.dot(w_ref[:, tap * cin:(tap + 1) * cin], win,
                                preferred_element_type=jnp.float32)
    o_ref[0] = acc.astype(o_ref.dtype)


def kernel(x_nchw, w2d, b2d):
    n, cin, h, w = x_nchw.shape
    cout = w2d.shape[0]
    ho, wo = h // 2, w // 2
    th = 16 if ho % 16 == 0 else ho
    tq = 8 if th % 8 == 0 else th
    nt = th * wo
    rt = ho // th

    pint, phalo = _pool_matrices(tq, wo)
    pint = pint.astype(jnp.bfloat16)
    phalo = phalo.astype(jnp.bfloat16)
    body = functools.partial(_pool_conv_kernel, cin=cin, th=th, tq=tq, wo=wo)
    out = pl.pallas_call(
        body,
        out_shape=jax.ShapeDtypeStruct((n, cout, ho * wo), x_nchw.dtype),
        grid=(n, rt),
        in_specs=[
            pl.BlockSpec((1, cin, h * w), lambda i, k: (i, 0, 0)),
            pl.BlockSpec((cout, 9 * cin), lambda i, k: (0, 0)),
            pl.BlockSpec((cout, 1), lambda i, k: (0, 0)),
            pl.BlockSpec(pint.shape, lambda i, k: (0, 0)),
            pl.BlockSpec(phalo.shape, lambda i, k: (0, 0)),
        ],
        out_specs=pl.BlockSpec((1, cout, nt), lambda i, k: (i, 0, k)),
        scratch_shapes=[
            pltpu.VMEM((cin, (th + 2) * wo + 2), jnp.bfloat16),
        ],
        compiler_params=pltpu.CompilerParams(
            dimension_semantics=("parallel", "parallel"),
            vmem_limit_bytes=48 * 1024 * 1024,
        ),
    )(x_nchw.astype(jnp.bfloat16).reshape(n, cin, h * w),
      w2d.astype(jnp.bfloat16),
      b2d.astype(jnp.float32), pint, phalo)
    return out.reshape(n, cout, ho, wo)


# speedup vs baseline: 21.8310x; 1.1713x over previous
import functools

import numpy as np
import jax
import jax.numpy as jnp
from jax import lax
from jax.experimental import pallas as pl
from jax.experimental.pallas import tpu as pltpu


def _pool_matrices(tq, wo):
    w2 = 2 * wo
    pint = np.zeros((2 * tq * w2, tq * wo), np.float32)
    for y in range(tq):
        for xo in range(wo):
            base = 2 * y * w2 + 2 * xo
            for l in (base, base + 1, base + w2, base + w2 + 1):
                pint[l, y * wo + xo] = 0.25
    phalo = np.zeros((2 * w2, wo), np.float32)
    for xo in range(wo):
        base = 2 * xo
        for l in (base, base + 1, base + w2, base + w2 + 1):
            phalo[l, xo] = 0.25
    return jnp.asarray(pint), jnp.asarray(phalo)


def _pool_conv_kernel(x_ref, w_ref, b_ref, pint_ref, phalo_ref, o_ref,
                      flat_ref, *, cin, th, tq, wo):
    hw = x_ref.shape[2]
    w2 = 2 * wo
    nt = th * wo
    nq = tq * wo
    lanes_t = 2 * th * w2
    lanes_q = 2 * tq * w2
    r = pl.program_id(1)
    nr = pl.num_programs(1)
    cdt = flat_ref.dtype

    s0 = pl.multiple_of(r * lanes_t, lanes_t)
    t0 = pl.multiple_of(jnp.maximum(s0 - 2 * w2, 0), 2 * w2)
    b0 = pl.multiple_of(jnp.minimum(s0 + lanes_t, hw - 2 * w2), 2 * w2)

    for q in range(th // tq):
        slab = x_ref[0, :, pl.ds(s0 + q * lanes_q, lanes_q)]
        part = jnp.dot(slab, pint_ref[...], preferred_element_type=jnp.float32)
        flat_ref[:, 1 + wo + q * nq:1 + wo + (q + 1) * nq] = part.astype(cdt)

    top = jnp.dot(x_ref[0, :, pl.ds(t0, 2 * w2)], phalo_ref[...],
                  preferred_element_type=jnp.float32)
    bot = jnp.dot(x_ref[0, :, pl.ds(b0, 2 * w2)], phalo_ref[...],
                  preferred_element_type=jnp.float32)
    top = top * (r > 0).astype(top.dtype)
    bot = bot * (r < nr - 1).astype(bot.dtype)

    zc = jnp.zeros((cin, 1), cdt)
    flat_ref[:, 0:1] = zc
    flat_ref[:, 1:1 + wo] = top.astype(cdt)
    flat_ref[:, 1 + wo + nt:1 + 2 * wo + nt] = bot.astype(cdt)
    flat_ref[:, 1 + 2 * wo + nt:2 + 2 * wo + nt] = zc

    nc = tq * wo
    colc = lax.broadcasted_iota(jnp.int32, (1, tq, wo), 2).reshape(1, nc)
    for c in range(th // tq):
        base = c * nc
        acc = b_ref[...] * jnp.ones((1, nc), jnp.float32)
        for kh in range(3):
            for kw in range(3):
                tap = kh * 3 + kw
                s = base + kh * wo + kw
                win = flat_ref[:, s:s + nc]
                if kw == 0:
                    win = jnp.where(colc != 0, win, jnp.zeros_like(win))
                elif kw == 2:
                    win = jnp.where(colc != wo - 1, win, jnp.zeros_like(win))
                acc = acc + jnp.dot(w_ref[:, tap * cin:(tap + 1) * cin], win,
                                    preferred_element_type=jnp.float32)
        o_ref[0, :, base:base + nc] = acc.astype(o_ref.dtype)


def kernel(x_nchw, w2d, b2d):
    n, cin, h, w = x_nchw.shape
    cout = w2d.shape[0]
    ho, wo = h // 2, w // 2
    th = 32 if ho % 32 == 0 else ho
    tq = 8 if th % 8 == 0 else th
    nt = th * wo
    rt = ho // th

    pint, phalo = _pool_matrices(tq, wo)
    pint = pint.astype(jnp.bfloat16)
    phalo = phalo.astype(jnp.bfloat16)
    body = functools.partial(_pool_conv_kernel, cin=cin, th=th, tq=tq, wo=wo)
    out = pl.pallas_call(
        body,
        out_shape=jax.ShapeDtypeStruct((n, cout, ho * wo), x_nchw.dtype),
        grid=(n, rt),
        in_specs=[
            pl.BlockSpec((1, cin, h * w), lambda i, k: (i, 0, 0)),
            pl.BlockSpec((cout, 9 * cin), lambda i, k: (0, 0)),
            pl.BlockSpec((cout, 1), lambda i, k: (0, 0)),
            pl.BlockSpec(pint.shape, lambda i, k: (0, 0)),
            pl.BlockSpec(phalo.shape, lambda i, k: (0, 0)),
        ],
        out_specs=pl.BlockSpec((1, cout, nt), lambda i, k: (i, 0, k)),
        scratch_shapes=[
            pltpu.VMEM((cin, (th + 2) * wo + 2), jnp.bfloat16),
        ],
        compiler_params=pltpu.CompilerParams(
            dimension_semantics=("parallel", "parallel"),
            vmem_limit_bytes=48 * 1024 * 1024,
        ),
    )(x_nchw.astype(jnp.bfloat16).reshape(n, cin, h * w),
      w2d.astype(jnp.bfloat16),
      b2d.astype(jnp.float32), pint, phalo)
    return out.reshape(n, cout, ho, wo)


# speedup vs baseline: 23.9213x; 1.0957x over previous
import functools

import numpy as np
import jax
import jax.numpy as jnp
from jax import lax
from jax.experimental import pallas as pl
from jax.experimental.pallas import tpu as pltpu


def _pool_matrix(tq, wo):
    w2 = 2 * wo
    pint = np.zeros((2 * tq * w2, tq * wo), np.float32)
    for y in range(tq):
        for xo in range(wo):
            base = 2 * y * w2 + 2 * xo
            for l in (base, base + 1, base + w2, base + w2 + 1):
                pint[l, y * wo + xo] = 0.25
    return jnp.asarray(pint)


def _pool_conv_kernel(x_ref, w_ref, b_ref, pint_ref, o_ref, flat_ref,
                      *, cin, ho, tq, wo, nimg):
    nt = ho * wo
    nq = tq * wo
    lanes_q = 2 * tq * 2 * wo
    cdt = jnp.bfloat16

    zc = jnp.zeros((cin, 1 + wo), cdt)
    for m in range(nimg):
        buf = flat_ref.at[m]
        for q in range(ho // tq):
            slab = x_ref[m, :, pl.ds(q * lanes_q, lanes_q)]
            part = jnp.dot(slab, pint_ref[...],
                           preferred_element_type=jnp.float32)
            buf[:, 1 + wo + q * nq:1 + wo + (q + 1) * nq] = part.astype(cdt)
        buf[:, 0:1 + wo] = zc
        buf[:, 1 + wo + nt:2 + 2 * wo + nt] = zc

    colc = lax.broadcasted_iota(jnp.int32, (1, tq, wo), 2).reshape(1, nq)
    for m in range(nimg):
        buf = flat_ref.at[m]
        for c in range(ho // tq):
            base = c * nq
            acc = b_ref[...] * jnp.ones((1, nq), jnp.float32)
            for kh in range(3):
                for kw in range(3):
                    tap = kh * 3 + kw
                    s = base + kh * wo + kw
                    win = buf[:, s:s + nq]
                    if kw == 0:
                        win = jnp.where(colc != 0, win, jnp.zeros_like(win))
                    elif kw == 2:
                        win = jnp.where(colc != wo - 1, win,
                                        jnp.zeros_like(win))
                    acc = acc + jnp.dot(w_ref[:, tap * cin:(tap + 1) * cin],
                                        win,
                                        preferred_element_type=jnp.float32)
            o_ref[m, :, base:base + nq] = acc.astype(o_ref.dtype)


def kernel(x_nchw, w2d, b2d):
    n, cin, h, w = x_nchw.shape
    cout = w2d.shape[0]
    ho, wo = h // 2, w // 2
    tq = 8 if ho % 8 == 0 else ho
    nimg = 4 if n % 4 == 0 else 1

    pint = _pool_matrix(tq, wo).astype(jnp.bfloat16)
    body = functools.partial(_pool_conv_kernel, cin=cin, ho=ho, tq=tq, wo=wo,
                             nimg=nimg)
    out = pl.pallas_call(
        body,
        out_shape=jax.ShapeDtypeStruct((n, cout, ho * wo), x_nchw.dtype),
        grid=(n // nimg,),
        in_specs=[
            pl.BlockSpec((nimg, cin, h * w), lambda i: (i, 0, 0)),
            pl.BlockSpec((cout, 9 * cin), lambda i: (0, 0)),
            pl.BlockSpec((cout, 1), lambda i: (0, 0)),
            pl.BlockSpec(pint.shape, lambda i: (0, 0)),
        ],
        out_specs=pl.BlockSpec((nimg, cout, ho * wo), lambda i: (i, 0, 0)),
        scratch_shapes=[
            pltpu.VMEM((nimg, cin, (ho + 2) * wo + 2), jnp.bfloat16),
        ],
        compiler_params=pltpu.CompilerParams(
            dimension_semantics=("arbitrary",),
            vmem_limit_bytes=48 * 1024 * 1024,
        ),
    )(x_nchw.astype(jnp.bfloat16).reshape(n, cin, h * w),
      w2d.astype(jnp.bfloat16),
      b2d.astype(jnp.float32), pint)
    return out.reshape(n, cout, ho, wo)
